# baseline (speedup 1.0000x reference)
import functools

import jax
import jax.numpy as jnp
from jax import lax
from jax.experimental import pallas as pl
from jax.experimental.pallas import tpu as pltpu
from jax.experimental.pallas import tpu_sc as plsc

NC, NS = 2, 16
NW = NC * NS
BLK = 80
HI = lax.Precision.HIGHEST


def _proj_body(x_ref, wa_ref, wb_ref, b1_ref, a_ref, b_ref):
    xb = x_ref[...]
    a_ref[...] = (jnp.dot(xb, wa_ref[...].T, precision=HI)
                  + b1_ref[...]).astype(jnp.bfloat16)
    b_ref[...] = jnp.dot(xb, wb_ref[...].T, precision=HI).astype(jnp.bfloat16)


def _node_proj(x2, Wa, Wb, b1):
    n, c = x2.shape
    bn = 1000
    return pl.pallas_call(
        _proj_body,
        grid=(n // bn,),
        in_specs=[
            pl.BlockSpec((bn, c), lambda i: (i, 0)),
            pl.BlockSpec(Wa.shape, lambda i: (0, 0)),
            pl.BlockSpec(Wb.shape, lambda i: (0, 0)),
            pl.BlockSpec((1, b1.shape[1]), lambda i: (0, 0)),
        ],
        out_specs=[
            pl.BlockSpec((bn, Wa.shape[0]), lambda i: (i, 0)),
            pl.BlockSpec((bn, Wb.shape[0]), lambda i: (i, 0)),
        ],
        out_shape=[
            jax.ShapeDtypeStruct((n, Wa.shape[0]), jnp.bfloat16),
            jax.ShapeDtypeStruct((n, Wb.shape[0]), jnp.bfloat16),
        ],
    )(x2, Wa, Wb, b1)


NSLOT = 4


def _gather_body(nb, a_hbm, b_hbm, rowi_hbm, coli_hbm, ga_hbm, gb_hbm,
                 idxr, idxc, bufa, bufb, sga, sgb, soa, sob):
    c = lax.axis_index("c")
    s = lax.axis_index("s")
    w = s * NC + c
    base = w * (nb * BLK)
    pltpu.sync_copy(rowi_hbm.at[w], idxr)
    pltpu.sync_copy(coli_hbm.at[w], idxc)

    def start_gather(j, b):
        pltpu.async_copy(a_hbm.at[idxr.at[j]], bufa.at[b], sga.at[b])
        pltpu.async_copy(b_hbm.at[idxc.at[j]], bufb.at[b], sgb.at[b])

    def wait_gather(b):
        pltpu.make_async_copy(ga_hbm.at[pl.ds(0, BLK)], bufa.at[b], sga.at[b]).wait()
        pltpu.make_async_copy(gb_hbm.at[pl.ds(0, BLK)], bufb.at[b], sgb.at[b]).wait()

    def start_out(j, b):
        off = base + j * BLK
        pltpu.async_copy(bufa.at[b], ga_hbm.at[pl.ds(off, BLK)], soa.at[b])
        pltpu.async_copy(bufb.at[b], gb_hbm.at[pl.ds(off, BLK)], sob.at[b])

    def wait_out(b):
        pltpu.make_async_copy(bufa.at[b], ga_hbm.at[pl.ds(0, BLK)], soa.at[b]).wait()
        pltpu.make_async_copy(bufb.at[b], gb_hbm.at[pl.ds(0, BLK)], sob.at[b]).wait()

    start_gather(0, 0)
    start_gather(1, 1)
    nbp = (nb + NSLOT - 1) // NSLOT * NSLOT

    @pl.loop(0, nbp, step=NSLOT)
    def _(j0):
        for b in range(NSLOT):
            j = j0 + b

            @pl.when(j < nb)
            def _():
                wait_gather(b)
                start_out(j, b)

            t = j + 2
            bt = (b + 2) % NSLOT

            @pl.when(jnp.logical_and(t >= NSLOT, t < nb))
            def _():
                wait_out(bt)

            @pl.when(t < nb)
            def _():
                start_gather(t, bt)

    for b in range(NSLOT):
        wait_out(b)


def _sc_gather(A, B, rowi, coli):
    nb = rowi.shape[1]
    e = NW * nb * BLK
    d = A.shape[1]
    mesh = plsc.VectorSubcoreMesh(core_axis_name="c", subcore_axis_name="s")
    kf = pl.kernel(
        functools.partial(_gather_body, nb),
        out_type=[jax.ShapeDtypeStruct((e, d), jnp.int32)] * 2,
        mesh=mesh,
        scratch_types=[
            pltpu.VMEM((nb, BLK), jnp.int32),
            pltpu.VMEM((nb, BLK), jnp.int32),
            pltpu.VMEM((NSLOT, BLK, d), jnp.int32),
            pltpu.VMEM((NSLOT, BLK, d), jnp.int32),
            pltpu.SemaphoreType.DMA((NSLOT,)),
            pltpu.SemaphoreType.DMA((NSLOT,)),
            pltpu.SemaphoreType.DMA((NSLOT,)),
            pltpu.SemaphoreType.DMA((NSLOT,)),
        ],
        compiler_params=pltpu.CompilerParams(use_tc_tiling_on_sc=False),
    )
    return kf(A, B, rowi, coli)


def _unpack_pair(g):
    lo = lax.bitcast_convert_type(jnp.left_shift(g, 16), jnp.float32)
    hi = lax.bitcast_convert_type(
        jnp.bitwise_and(g, jnp.int32(-65536)), jnp.float32)
    return lo, hi


def _edge_body(ga_ref, gb_ref, p0_ref, p1_ref, w20_ref, b20_ref,
               w21_ref, b21_ref, m_ref):
    a0, a1 = _unpack_pair(ga_ref[...])
    b0, b1 = _unpack_pair(gb_ref[...])
    h0 = jnp.maximum(a0 + b0, 0.0)
    h1 = jnp.maximum(a1 + b1, 0.0)
    m0 = jnp.maximum(jnp.dot(h0, w20_ref[...].T, precision=HI) + b20_ref[...], 0.0)
    m1 = jnp.maximum(jnp.dot(h1, w21_ref[...].T, precision=HI) + b21_ref[...], 0.0)
    m_ref[...] = m0 * p0_ref[...] + m1 * p1_ref[...]


def _edge_mlp(GA, GB, p0, p1, W20, b20, W21, b21):
    e, d = GA.shape
    h = W20.shape[0]
    be = 3200
    return pl.pallas_call(
        _edge_body,
        grid=(e // be,),
        in_specs=[
            pl.BlockSpec((be, d), lambda i: (i, 0)),
            pl.BlockSpec((be, d), lambda i: (i, 0)),
            pl.BlockSpec((be, 1), lambda i: (i, 0)),
            pl.BlockSpec((be, 1), lambda i: (i, 0)),
            pl.BlockSpec(W20.shape, lambda i: (0, 0)),
            pl.BlockSpec((1, h), lambda i: (0, 0)),
            pl.BlockSpec(W21.shape, lambda i: (0, 0)),
            pl.BlockSpec((1, h), lambda i: (0, 0)),
        ],
        out_specs=pl.BlockSpec((be, h), lambda i: (i, 0)),
        out_shape=jax.ShapeDtypeStruct((e, h), jnp.float32),
    )(GA, GB, p0, p1, W20, b20, W21, b21)


def _scatter_body(nb, n, m_hbm, rowi_hbm, zer_hbm, out_hbm,
                  ib0, ib1, ib2, ib3, mbuf, smi, smm, sms, acc):
    c = lax.axis_index("c")
    s = lax.axis_index("s")
    w = s * NC + c
    base = w * (nb * BLK)
    nsl = n // NS
    ibs = [ib0, ib1, ib2, ib3]
    pltpu.sync_copy(zer_hbm.at[pl.ds(s * nsl, nsl)], acc.at[pl.ds(s * nsl, nsl)])
    plsc.subcore_barrier()

    def start_load(j, b):
        pltpu.async_copy(rowi_hbm.at[w].at[j], ibs[b], smi.at[b])
        pltpu.async_copy(m_hbm.at[pl.ds(base + j * BLK, BLK)], mbuf.at[b],
                         smm.at[b])

    def wait_load(b):
        pltpu.make_async_copy(rowi_hbm.at[w].at[0], ibs[b], smi.at[b]).wait()
        pltpu.make_async_copy(m_hbm.at[pl.ds(0, BLK)], mbuf.at[b],
                              smm.at[b]).wait()

    def start_scat(b):
        pltpu.async_copy(mbuf.at[b], acc.at[ibs[b]], sms.at[b], add=True)

    def wait_scat(b):
        pltpu.make_async_copy(mbuf.at[b], acc.at[ibs[b]], sms.at[b]).wait()

    start_load(0, 0)
    start_load(1, 1)
    nbp = (nb + NSLOT - 1) // NSLOT * NSLOT

    @pl.loop(0, nbp, step=NSLOT)
    def _(j0):
        for b in range(NSLOT):
            j = j0 + b

            @pl.when(j < nb)
            def _():
                wait_load(b)
                start_scat(b)

            t = j + 2
            bt = (b + 2) % NSLOT

            @pl.when(jnp.logical_and(t >= NSLOT, t < nb))
            def _():
                wait_scat(bt)

            @pl.when(t < nb)
            def _():
                start_load(t, bt)

    for b in range(NSLOT):
        wait_scat(b)
    plsc.subcore_barrier()
    pltpu.sync_copy(acc.at[pl.ds(s * nsl, nsl)],
                    out_hbm.at[c, pl.ds(s * nsl, nsl)])


def _sc_scatter(M, rowi, zer):
    nb = rowi.shape[1]
    n, h = zer.shape
    mesh = plsc.VectorSubcoreMesh(core_axis_name="c", subcore_axis_name="s")
    kf = pl.kernel(
        functools.partial(_scatter_body, nb, n),
        out_type=jax.ShapeDtypeStruct((NC, n, h), jnp.float32),
        mesh=mesh,
        scratch_types=[
            pltpu.VMEM((BLK,), jnp.int32),
            pltpu.VMEM((BLK,), jnp.int32),
            pltpu.VMEM((BLK,), jnp.int32),
            pltpu.VMEM((BLK,), jnp.int32),
            pltpu.VMEM((NSLOT, BLK, h), jnp.float32),
            pltpu.SemaphoreType.DMA((NSLOT,)),
            pltpu.SemaphoreType.DMA((NSLOT,)),
            pltpu.SemaphoreType.DMA((NSLOT,)),
            pltpu.VMEM_SHARED((n, h), jnp.float32),
        ],
        compiler_params=pltpu.CompilerParams(use_tc_tiling_on_sc=False),
    )
    return kf(M, rowi, zer)


def _out_body(x_ref, p_ref, w1_ref, b1_ref, w2_ref, b2_ref, w3_ref, b3_ref,
              o_ref):
    xb = x_ref[...]
    agg = p_ref[0] + p_ref[1]
    w1 = w1_ref[...]
    cdim = x_ref.shape[1]
    a1 = (jnp.dot(xb, w1[:, :cdim].T, precision=HI)
          + jnp.dot(agg, w1[:, cdim:].T, precision=HI) + b1_ref[...])
    hh = jnp.maximum(a1, 0.0)
    hh = jnp.maximum(jnp.dot(hh, w2_ref[...].T, precision=HI) + b2_ref[...], 0.0)
    o_ref[...] = jnp.dot(hh, w3_ref[...].T, precision=HI) + b3_ref[...] + xb


def _out_mlp(x2, P, W1, b1, W2, b2, W3, b3):
    n, c = x2.shape
    h = W2.shape[0]
    bn = 1000
    return pl.pallas_call(
        _out_body,
        grid=(n // bn,),
        in_specs=[
            pl.BlockSpec((bn, c), lambda i: (i, 0)),
            pl.BlockSpec((NC, bn, h), lambda i: (0, i, 0)),
            pl.BlockSpec(W1.shape, lambda i: (0, 0)),
            pl.BlockSpec((1, h), lambda i: (0, 0)),
            pl.BlockSpec(W2.shape, lambda i: (0, 0)),
            pl.BlockSpec((1, h), lambda i: (0, 0)),
            pl.BlockSpec(W3.shape, lambda i: (0, 0)),
            pl.BlockSpec((1, c), lambda i: (0, 0)),
        ],
        out_specs=pl.BlockSpec((bn, c), lambda i: (i, 0)),
        out_shape=jax.ShapeDtypeStruct((n, c), jnp.float32),
    )(x2, P, W1, b1, W2, b2, W3, b3)


def kernel(x, edge_index, edge_prob, W_msg1_0, b_msg1_0, W_msg1_1, b_msg1_1,
           W_msg2_0, b_msg2_0, W_msg2_1, b_msg2_1, W_out1, b_out1, W_out2,
           b_out2, W_out3, b_out3):
    _, n, c = x.shape
    e = edge_index.shape[1]
    h = W_msg2_0.shape[0]
    assert e % (NW * BLK) == 0 and n % NS == 0
    nb = e // (NW * BLK)

    x2 = x[0]
    perm = jnp.stack([jnp.arange(h), jnp.arange(h) + h], axis=1).reshape(-1)
    Wa = jnp.concatenate([W_msg1_0[:, :c], W_msg1_1[:, :c]], axis=0)[perm]
    Wb = jnp.concatenate([W_msg1_0[:, c:], W_msg1_1[:, c:]], axis=0)[perm]
    b1c = jnp.concatenate([b_msg1_0, b_msg1_1])[perm].reshape(1, 2 * h)

    A, B = _node_proj(x2, Wa, Wb, b1c)
    A32 = lax.bitcast_convert_type(A.reshape(n, h, 2), jnp.int32)
    B32 = lax.bitcast_convert_type(B.reshape(n, h, 2), jnp.int32)

    rowi = edge_index[0].reshape(NW, nb, BLK)
    coli = edge_index[1].reshape(NW, nb, BLK)
    GA, GB = _sc_gather(A32, B32, rowi, coli)

    p0 = edge_prob[0].reshape(e, 1)
    p1 = edge_prob[1].reshape(e, 1)
    M = _edge_mlp(GA, GB, p0, p1, W_msg2_0, b_msg2_0.reshape(1, -1),
                  W_msg2_1, b_msg2_1.reshape(1, -1))

    n_pad = ((n + 8 * NS - 1) // (8 * NS)) * (8 * NS)
    zer = jnp.zeros((n_pad, h), jnp.float32)
    P = _sc_scatter(M, rowi, zer)

    out = _out_mlp(x2, P, W_out1, b_out1.reshape(1, -1), W_out2,
                   b_out2.reshape(1, -1), W_out3, b_out3.reshape(1, -1))
    return out[None]


# speedup vs baseline: 1.3402x; 1.3402x over previous
import functools

import jax
import jax.numpy as jnp
from jax import lax
from jax.experimental import pallas as pl
from jax.experimental.pallas import tpu as pltpu
from jax.experimental.pallas import tpu_sc as plsc

NC, NS = 2, 16
NW = NC * NS
BLK = 80
HI = lax.Precision.HIGHEST


def _proj_body(x_ref, wa_ref, wb_ref, b1_ref, a_ref, b_ref):
    xb = x_ref[...]
    a_ref[...] = jnp.dot(xb, wa_ref[...].T, precision=HI) + b1_ref[...]
    b_ref[...] = jnp.dot(xb, wb_ref[...].T, precision=HI)


def _node_proj(x2, Wa, Wb, b1):
    n, c = x2.shape
    bn = 1000
    return pl.pallas_call(
        _proj_body,
        grid=(n // bn,),
        in_specs=[
            pl.BlockSpec((bn, c), lambda i: (i, 0)),
            pl.BlockSpec(Wa.shape, lambda i: (0, 0)),
            pl.BlockSpec(Wb.shape, lambda i: (0, 0)),
            pl.BlockSpec((1, b1.shape[1]), lambda i: (0, 0)),
        ],
        out_specs=[
            pl.BlockSpec((bn, Wa.shape[0]), lambda i: (i, 0)),
            pl.BlockSpec((bn, Wb.shape[0]), lambda i: (i, 0)),
        ],
        out_shape=[
            jax.ShapeDtypeStruct((n, Wa.shape[0]), jnp.float32),
            jax.ShapeDtypeStruct((n, Wb.shape[0]), jnp.float32),
        ],
    )(x2, Wa, Wb, b1)


NSLOT = 4


def _gather_body(nb, d, a_hbm, b_hbm, rowi_hbm, coli_hbm, g_hbm,
                 idxr, idxc, bufa, bufb, sga, sgb, so):
    c = lax.axis_index("c")
    s = lax.axis_index("s")
    w = s * NC + c
    base = w * (nb * BLK)
    pltpu.sync_copy(rowi_hbm.at[w], idxr)
    pltpu.sync_copy(coli_hbm.at[w], idxc)
    nk = d // 16

    def start_gather(j, b):
        pltpu.async_copy(a_hbm.at[idxr.at[j]], bufa.at[b], sga.at[b])
        pltpu.async_copy(b_hbm.at[idxc.at[j]], bufb.at[b], sgb.at[b])

    def wait_gather(b):
        pltpu.make_async_copy(g_hbm.at[pl.ds(0, BLK)], bufa.at[b], sga.at[b]).wait()
        pltpu.make_async_copy(g_hbm.at[pl.ds(0, BLK)], bufb.at[b], sgb.at[b]).wait()

    def add_rows(b):
        ba = bufa.at[b]
        bb = bufb.at[b]

        @pl.loop(0, BLK)
        def _(r):
            for k in range(nk):
                sl = pl.ds(k * 16, 16)
                ba[r, sl] = ba[r, sl] + bb[r, sl]

    def start_out(j, b):
        off = base + j * BLK
        pltpu.async_copy(bufa.at[b], g_hbm.at[pl.ds(off, BLK)], so.at[b])

    def wait_out(b):
        pltpu.make_async_copy(bufa.at[b], g_hbm.at[pl.ds(0, BLK)], so.at[b]).wait()

    start_gather(0, 0)
    start_gather(1, 1)
    nbp = (nb + NSLOT - 1) // NSLOT * NSLOT

    @pl.loop(0, nbp, step=NSLOT)
    def _(j0):
        for b in range(NSLOT):
            j = j0 + b

            @pl.when(j < nb)
            def _():
                wait_gather(b)
                add_rows(b)
                start_out(j, b)

            t = j + 2
            bt = (b + 2) % NSLOT

            @pl.when(jnp.logical_and(t >= NSLOT, t < nb))
            def _():
                wait_out(bt)

            @pl.when(t < nb)
            def _():
                start_gather(t, bt)

    for b in range(NSLOT):
        wait_out(b)


def _sc_gather(A, B, rowi, coli):
    nb = rowi.shape[1]
    e = NW * nb * BLK
    d = A.shape[1]
    mesh = plsc.VectorSubcoreMesh(core_axis_name="c", subcore_axis_name="s")
    kf = pl.kernel(
        functools.partial(_gather_body, nb, d),
        out_type=jax.ShapeDtypeStruct((e, d), jnp.float32),
        mesh=mesh,
        scratch_types=[
            pltpu.VMEM((nb, BLK), jnp.int32),
            pltpu.VMEM((nb, BLK), jnp.int32),
            pltpu.VMEM((NSLOT, BLK, d), jnp.float32),
            pltpu.VMEM((NSLOT, BLK, d), jnp.float32),
            pltpu.SemaphoreType.DMA((NSLOT,)),
            pltpu.SemaphoreType.DMA((NSLOT,)),
            pltpu.SemaphoreType.DMA((NSLOT,)),
        ],
    )
    return kf(A, B, rowi, coli)


def _edge_body(g_ref, p0_ref, p1_ref, w20_ref, b20_ref,
               w21_ref, b21_ref, m_ref):
    h = jnp.maximum(g_ref[...], 0.0)
    hdim = w20_ref.shape[1]
    h0 = h[:, :hdim]
    h1 = h[:, hdim:]
    m0 = jnp.maximum(jnp.dot(h0, w20_ref[...].T, precision=HI) + b20_ref[...], 0.0)
    m1 = jnp.maximum(jnp.dot(h1, w21_ref[...].T, precision=HI) + b21_ref[...], 0.0)
    m_ref[...] = m0 * p0_ref[...] + m1 * p1_ref[...]


def _edge_mlp(G, p0, p1, W20, b20, W21, b21):
    e, d = G.shape
    h = W20.shape[0]
    be = 3200
    return pl.pallas_call(
        _edge_body,
        grid=(e // be,),
        in_specs=[
            pl.BlockSpec((be, d), lambda i: (i, 0)),
            pl.BlockSpec((be, 1), lambda i: (i, 0)),
            pl.BlockSpec((be, 1), lambda i: (i, 0)),
            pl.BlockSpec(W20.shape, lambda i: (0, 0)),
            pl.BlockSpec((1, h), lambda i: (0, 0)),
            pl.BlockSpec(W21.shape, lambda i: (0, 0)),
            pl.BlockSpec((1, h), lambda i: (0, 0)),
        ],
        out_specs=pl.BlockSpec((be, h), lambda i: (i, 0)),
        out_shape=jax.ShapeDtypeStruct((e, h), jnp.float32),
    )(G, p0, p1, W20, b20, W21, b21)


def _scatter_body(nb, n, m_hbm, rowi_hbm, zer_hbm, out_hbm,
                  ib0, ib1, ib2, ib3, mbuf, smi, smm, sms, acc):
    c = lax.axis_index("c")
    s = lax.axis_index("s")
    w = s * NC + c
    base = w * (nb * BLK)
    nsl = n // NS
    ibs = [ib0, ib1, ib2, ib3]
    pltpu.sync_copy(zer_hbm.at[pl.ds(s * nsl, nsl)], acc.at[pl.ds(s * nsl, nsl)])
    plsc.subcore_barrier()

    def start_load(j, b):
        pltpu.async_copy(rowi_hbm.at[w].at[j], ibs[b], smi.at[b])
        pltpu.async_copy(m_hbm.at[pl.ds(base + j * BLK, BLK)], mbuf.at[b],
                         smm.at[b])

    def wait_load(b):
        pltpu.make_async_copy(rowi_hbm.at[w].at[0], ibs[b], smi.at[b]).wait()
        pltpu.make_async_copy(m_hbm.at[pl.ds(0, BLK)], mbuf.at[b],
                              smm.at[b]).wait()

    def start_scat(b):
        pltpu.async_copy(mbuf.at[b], acc.at[ibs[b]], sms.at[b], add=True)

    def wait_scat(b):
        pltpu.make_async_copy(mbuf.at[b], acc.at[ibs[b]], sms.at[b]).wait()

    start_load(0, 0)
    start_load(1, 1)
    nbp = (nb + NSLOT - 1) // NSLOT * NSLOT

    @pl.loop(0, nbp, step=NSLOT)
    def _(j0):
        for b in range(NSLOT):
            j = j0 + b

            @pl.when(j < nb)
            def _():
                wait_load(b)
                start_scat(b)

            t = j + 2
            bt = (b + 2) % NSLOT

            @pl.when(jnp.logical_and(t >= NSLOT, t < nb))
            def _():
                wait_scat(bt)

            @pl.when(t < nb)
            def _():
                start_load(t, bt)

    for b in range(NSLOT):
        wait_scat(b)
    plsc.subcore_barrier()
    pltpu.sync_copy(acc.at[pl.ds(s * nsl, nsl)],
                    out_hbm.at[c, pl.ds(s * nsl, nsl)])


def _sc_scatter(M, rowi, zer):
    nb = rowi.shape[1]
    n, h = zer.shape
    mesh = plsc.VectorSubcoreMesh(core_axis_name="c", subcore_axis_name="s")
    kf = pl.kernel(
        functools.partial(_scatter_body, nb, n),
        out_type=jax.ShapeDtypeStruct((NC, n, h), jnp.float32),
        mesh=mesh,
        scratch_types=[
            pltpu.VMEM((BLK,), jnp.int32),
            pltpu.VMEM((BLK,), jnp.int32),
            pltpu.VMEM((BLK,), jnp.int32),
            pltpu.VMEM((BLK,), jnp.int32),
            pltpu.VMEM((NSLOT, BLK, h), jnp.float32),
            pltpu.SemaphoreType.DMA((NSLOT,)),
            pltpu.SemaphoreType.DMA((NSLOT,)),
            pltpu.SemaphoreType.DMA((NSLOT,)),
            pltpu.VMEM_SHARED((n, h), jnp.float32),
        ],
        compiler_params=pltpu.CompilerParams(use_tc_tiling_on_sc=False),
    )
    return kf(M, rowi, zer)


def _out_body(x_ref, p_ref, w1_ref, b1_ref, w2_ref, b2_ref, w3_ref, b3_ref,
              o_ref):
    xb = x_ref[...]
    agg = p_ref[0] + p_ref[1]
    w1 = w1_ref[...]
    cdim = x_ref.shape[1]
    a1 = (jnp.dot(xb, w1[:, :cdim].T, precision=HI)
          + jnp.dot(agg, w1[:, cdim:].T, precision=HI) + b1_ref[...])
    hh = jnp.maximum(a1, 0.0)
    hh = jnp.maximum(jnp.dot(hh, w2_ref[...].T, precision=HI) + b2_ref[...], 0.0)
    o_ref[...] = jnp.dot(hh, w3_ref[...].T, precision=HI) + b3_ref[...] + xb


def _out_mlp(x2, P, W1, b1, W2, b2, W3, b3):
    n, c = x2.shape
    h = W2.shape[0]
    bn = 1000
    return pl.pallas_call(
        _out_body,
        grid=(n // bn,),
        in_specs=[
            pl.BlockSpec((bn, c), lambda i: (i, 0)),
            pl.BlockSpec((NC, bn, h), lambda i: (0, i, 0)),
            pl.BlockSpec(W1.shape, lambda i: (0, 0)),
            pl.BlockSpec((1, h), lambda i: (0, 0)),
            pl.BlockSpec(W2.shape, lambda i: (0, 0)),
            pl.BlockSpec((1, h), lambda i: (0, 0)),
            pl.BlockSpec(W3.shape, lambda i: (0, 0)),
            pl.BlockSpec((1, c), lambda i: (0, 0)),
        ],
        out_specs=pl.BlockSpec((bn, c), lambda i: (i, 0)),
        out_shape=jax.ShapeDtypeStruct((n, c), jnp.float32),
    )(x2, P, W1, b1, W2, b2, W3, b3)


def kernel(x, edge_index, edge_prob, W_msg1_0, b_msg1_0, W_msg1_1, b_msg1_1,
           W_msg2_0, b_msg2_0, W_msg2_1, b_msg2_1, W_out1, b_out1, W_out2,
           b_out2, W_out3, b_out3):
    _, n, c = x.shape
    e = edge_index.shape[1]
    h = W_msg2_0.shape[0]
    assert e % (NW * BLK) == 0 and n % NS == 0
    nb = e // (NW * BLK)

    x2 = x[0]
    Wa = jnp.concatenate([W_msg1_0[:, :c], W_msg1_1[:, :c]], axis=0)
    Wb = jnp.concatenate([W_msg1_0[:, c:], W_msg1_1[:, c:]], axis=0)
    b1c = jnp.concatenate([b_msg1_0, b_msg1_1]).reshape(1, 2 * h)

    A, B = _node_proj(x2, Wa, Wb, b1c)

    rowi = edge_index[0].reshape(NW, nb, BLK)
    coli = edge_index[1].reshape(NW, nb, BLK)
    G = _sc_gather(A, B, rowi, coli)

    p0 = edge_prob[0].reshape(e, 1)
    p1 = edge_prob[1].reshape(e, 1)
    M = _edge_mlp(G, p0, p1, W_msg2_0, b_msg2_0.reshape(1, -1),
                  W_msg2_1, b_msg2_1.reshape(1, -1))

    n_pad = ((n + 8 * NS - 1) // (8 * NS)) * (8 * NS)
    zer = jnp.zeros((n_pad, h), jnp.float32)
    P = _sc_scatter(M, rowi, zer)

    out = _out_mlp(x2, P, W_out1, b_out1.reshape(1, -1), W_out2,
                   b_out2.reshape(1, -1), W_out3, b_out3.reshape(1, -1))
    return out[None]
